# baseline (speedup 1.0000x reference)
import functools

import numpy as np
import jax
import jax.numpy as jnp
from jax import lax
from jax.experimental import pallas as pl
from jax.experimental.pallas import tpu as pltpu
from jax.experimental.pallas import tpu_sc as plsc

H = 0.001
Q = 3
N_COLS = 1024
N_XS = 16384

SC_ROWS = 6144
TC_ROWS = N_XS - SC_ROWS
BR = 512

NC, NS, L = 2, 16, 16
NW = NC * NS
RPW = SC_ROWS // NW
CR = 32
NCHUNK = RPW // CR
GPC = CR // L

_XE_OFF = [float(np.float32(H) * np.float32(Q - j)) for j in range(Q + 1)]

_mesh = plsc.VectorSubcoreMesh(
    core_axis_name="c", subcore_axis_name="s", num_cores=NC, num_subcores=NS
)


@functools.partial(
    pl.kernel,
    out_type=jax.ShapeDtypeStruct((SC_ROWS, N_COLS), jnp.float32),
    mesh=_mesh,
    scratch_types=[
        pltpu.VMEM((RPW,), jnp.float32),
        pltpu.VMEM((2 * L,), jnp.float32),
        pltpu.VMEM((CR, N_COLS), jnp.float32),
        pltpu.VMEM((CR, N_COLS), jnp.float32),
        pltpu.SemaphoreType.DMA,
        pltpu.SemaphoreType.DMA,
    ],
    compiler_params=pltpu.CompilerParams(needs_layout_passes=False),
)
def _sc_band(xs_hbm, b_hbm, out_hbm, xs_v, b_v, buf0, buf1, sem0, sem1):
    wid = lax.axis_index("s") * NC + lax.axis_index("c")
    row0 = wid * RPW
    pltpu.sync_copy(xs_hbm.at[pl.ds(row0, RPW)], xs_v)
    pltpu.sync_copy(b_hbm, b_v.at[pl.ds(0, L)])
    pltpu.sync_copy(b_hbm, b_v.at[pl.ds(L, L)])

    lanes = lax.iota(jnp.int32, L)
    coefs = [
        [plsc.load_gather(b_v, [jnp.full((L,), L + 4 * j + p, jnp.int32)])
         for p in range(Q + 1)]
        for j in range(Q + 1)
    ]
    zero16 = jnp.zeros((L,), jnp.float32)
    bufs = (buf0, buf1)
    sems = (sem0, sem1)

    def zero_buf(buf):
        @pl.loop(0, CR)
        def _(r):
            @pl.loop(0, N_COLS // L)
            def _(ci):
                buf[r, pl.ds(ci * L, L)] = zero16

    def group_first_i(c, g):
        off = pl.multiple_of(c * CR + g * L, L)
        x = xs_v[pl.ds(off, L)]
        fi = (x / H).astype(jnp.int32)
        return x, fi

    def fill(buf, c):
        for g in range(GPC):
            x, fi = group_first_i(c, g)
            xm = x - fi.astype(jnp.float32) * H
            rows = g * L + lanes
            for j in range(Q + 1):
                xe = xm + _XE_OFF[j]
                cj = coefs[j]
                v = ((cj[3] * xe + cj[2]) * xe + cj[1]) * xe + cj[0]
                plsc.store_scatter(buf, [rows, fi + j], v)

    def unscatter(buf, c):
        for g in range(GPC):
            _, fi = group_first_i(c, g)
            rows = g * L + lanes
            for j in range(Q + 1):
                plsc.store_scatter(buf, [rows, fi + j], zero16)

    def start_dma(s, c):
        return pltpu.async_copy(
            bufs[s], out_hbm.at[pl.ds(row0 + c * CR, CR)], sems[s]
        )

    zero_buf(buf0)
    fill(buf0, 0)
    start_dma(0, 0)
    zero_buf(buf1)
    fill(buf1, 1)
    start_dma(1, 1)

    @pl.loop(1, NCHUNK // 2)
    def _(cc):
        for s in range(2):
            c = cc * 2 + s
            pltpu.make_async_copy(
                bufs[s], out_hbm.at[pl.ds(row0 + (c - 2) * CR, CR)], sems[s]
            ).wait()
            unscatter(bufs[s], c - 2)
            fill(bufs[s], c)
            start_dma(s, c)

    for s in range(2):
        pltpu.make_async_copy(
            bufs[s], out_hbm.at[pl.ds(row0 + (NCHUNK - 2 + s) * CR, CR)], sems[s]
        ).wait()


def _tc_body(xs_ref, b_ref, out_ref):
    x = xs_ref[...]
    fi = (x / H).astype(jnp.int32)
    xm = x - fi.astype(jnp.float32) * H
    col = lax.broadcasted_iota(jnp.int32, (BR, N_COLS), 1)
    delta = col - fi
    acc = jnp.zeros((BR, N_COLS), jnp.float32)
    for j in range(Q + 1):
        xe = xm + _XE_OFF[j]
        v = ((b_ref[j, 3] * xe + b_ref[j, 2]) * xe + b_ref[j, 1]) * xe + b_ref[j, 0]
        acc = jnp.where(delta == j, v, acc)
    out_ref[...] = acc


@jax.jit
def kernel(xs, B):
    sc_part = _sc_band(xs, B.reshape(-1))
    tc_part = pl.pallas_call(
        _tc_body,
        grid=(TC_ROWS // BR,),
        in_specs=[
            pl.BlockSpec((BR, 1), lambda i: (i + SC_ROWS // BR, 0)),
            pl.BlockSpec(memory_space=pltpu.SMEM),
        ],
        out_specs=pl.BlockSpec((BR, N_COLS), lambda i: (i, 0)),
        out_shape=jax.ShapeDtypeStruct((TC_ROWS, N_COLS), jnp.float32),
        compiler_params=pltpu.CompilerParams(
            dimension_semantics=("parallel",),
        ),
    )(xs.reshape(N_XS, 1), B)
    return jnp.concatenate([sc_part, tc_part], axis=0)


# speedup vs baseline: 1.6998x; 1.6998x over previous
import functools

import numpy as np
import jax
import jax.numpy as jnp
from jax import lax
from jax.experimental import pallas as pl
from jax.experimental.pallas import tpu as pltpu
from jax.experimental.pallas import tpu_sc as plsc

H = 0.001
Q = 3
N_COLS = 1024
N_XS = 16384

NC, NS, L = 2, 16, 16
NW = NC * NS
RPW = N_XS // NW
CR = 32
NCHUNK = RPW // CR
GPC = CR // L

_XE_OFF = [float(np.float32(H) * np.float32(Q - j)) for j in range(Q + 1)]

_mesh = plsc.VectorSubcoreMesh(
    core_axis_name="c", subcore_axis_name="s", num_cores=NC, num_subcores=NS
)


@functools.partial(
    pl.kernel,
    out_type=jax.ShapeDtypeStruct((N_XS, N_COLS), jnp.float32),
    mesh=_mesh,
    scratch_types=[
        pltpu.VMEM((RPW,), jnp.float32),
        pltpu.VMEM((2 * L,), jnp.float32),
        pltpu.VMEM((CR, N_COLS), jnp.float32),
        pltpu.VMEM((CR, N_COLS), jnp.float32),
        pltpu.SemaphoreType.DMA,
        pltpu.SemaphoreType.DMA,
    ],
    compiler_params=pltpu.CompilerParams(needs_layout_passes=False),
)
def _sc_band(xs_hbm, b_hbm, out_hbm, xs_v, b_v, buf0, buf1, sem0, sem1):
    wid = lax.axis_index("s") * NC + lax.axis_index("c")
    row0 = wid * RPW
    xs_cp = pltpu.async_copy(xs_hbm.at[pl.ds(row0, RPW)], xs_v, sem0)
    b_cp0 = pltpu.async_copy(b_hbm, b_v.at[pl.ds(0, L)], sem1)
    b_cp1 = pltpu.async_copy(b_hbm, b_v.at[pl.ds(L, L)], sem1)

    lanes = lax.iota(jnp.int32, L)
    zero16 = jnp.zeros((L,), jnp.float32)
    bufs = (buf0, buf1)
    sems = (sem0, sem1)

    def zero_buf(buf):
        @pl.loop(0, CR)
        def _(r):
            @pl.loop(0, N_COLS // L)
            def _(ci):
                buf[r, pl.ds(ci * L, L)] = zero16

    zero_buf(buf0)
    zero_buf(buf1)
    xs_cp.wait()
    b_cp0.wait()
    b_cp1.wait()

    coefs = [
        [plsc.load_gather(b_v, [jnp.full((L,), L + 4 * j + p, jnp.int32)])
         for p in range(Q + 1)]
        for j in range(Q + 1)
    ]

    def group_first_i(c, g):
        off = pl.multiple_of(c * CR + g * L, L)
        x = xs_v[pl.ds(off, L)]
        fi = (x / H).astype(jnp.int32)
        return x, fi

    def fill(buf, c):
        for g in range(GPC):
            x, fi = group_first_i(c, g)
            xm = x - fi.astype(jnp.float32) * H
            rows = g * L + lanes
            for j in range(Q + 1):
                xe = xm + _XE_OFF[j]
                cj = coefs[j]
                v = ((cj[3] * xe + cj[2]) * xe + cj[1]) * xe + cj[0]
                plsc.store_scatter(buf, [rows, fi + j], v)

    def unscatter(buf, c):
        for g in range(GPC):
            _, fi = group_first_i(c, g)
            rows = g * L + lanes
            for j in range(Q + 1):
                plsc.store_scatter(buf, [rows, fi + j], zero16)

    def start_dma(s, c):
        return pltpu.async_copy(
            bufs[s], out_hbm.at[pl.ds(row0 + c * CR, CR)], sems[s]
        )

    fill(buf0, 0)
    start_dma(0, 0)
    fill(buf1, 1)
    start_dma(1, 1)

    @pl.loop(1, NCHUNK // 2)
    def _(cc):
        for s in range(2):
            c = cc * 2 + s
            pltpu.make_async_copy(
                bufs[s], out_hbm.at[pl.ds(row0 + (c - 2) * CR, CR)], sems[s]
            ).wait()
            unscatter(bufs[s], c - 2)
            fill(bufs[s], c)
            start_dma(s, c)

    for s in range(2):
        pltpu.make_async_copy(
            bufs[s], out_hbm.at[pl.ds(row0 + (NCHUNK - 2 + s) * CR, CR)], sems[s]
        ).wait()


@jax.jit
def kernel(xs, B):
    return _sc_band(xs, B.reshape(-1))


# speedup vs baseline: 2.3266x; 1.3687x over previous
import functools

import numpy as np
import jax
import jax.numpy as jnp
from jax import lax
from jax.experimental import pallas as pl
from jax.experimental.pallas import tpu as pltpu
from jax.experimental.pallas import tpu_sc as plsc

H = 0.001
Q = 3
N_COLS = 1024
N_XS = 16384

NC, NS, L = 2, 16, 16
NW = NC * NS
RPW = N_XS // NW
CR = 32
NCHUNK = RPW // CR
GPC = CR // L

_XE_OFF = [float(np.float32(H) * np.float32(Q - j)) for j in range(Q + 1)]

_mesh = plsc.VectorSubcoreMesh(
    core_axis_name="c", subcore_axis_name="s", num_cores=NC, num_subcores=NS
)


@functools.partial(
    pl.kernel,
    out_type=jax.ShapeDtypeStruct((N_XS, N_COLS), jnp.float32),
    mesh=_mesh,
    scratch_types=[
        pltpu.VMEM((RPW,), jnp.float32),
        pltpu.VMEM((2 * L,), jnp.float32),
        pltpu.VMEM((CR, N_COLS), jnp.float32),
        pltpu.VMEM((CR, N_COLS), jnp.float32),
        pltpu.SemaphoreType.DMA,
        pltpu.SemaphoreType.DMA,
    ],
    compiler_params=pltpu.CompilerParams(needs_layout_passes=False),
)
def _sc_band(xs_hbm, b_hbm, out_hbm, xs_v, b_v, buf0, buf1, sem0, sem1):
    wid = lax.axis_index("s") * NC + lax.axis_index("c")
    row0 = wid * RPW
    xs_cp = pltpu.async_copy(xs_hbm.at[pl.ds(row0, RPW)], xs_v, sem0)
    b_cp0 = pltpu.async_copy(b_hbm, b_v.at[pl.ds(0, L)], sem1)
    b_cp1 = pltpu.async_copy(b_hbm, b_v.at[pl.ds(L, L)], sem1)

    lanes = lax.iota(jnp.int32, L)
    zero16 = jnp.zeros((L,), jnp.float32)
    bufs = (buf0, buf1)
    sems = (sem0, sem1)

    def zero_buf(buf):
        @pl.loop(0, CR)
        def _(r):
            for ci in range(N_COLS // L):
                buf[r, pl.ds(ci * L, L)] = zero16

    zero_buf(buf0)
    xs_cp.wait()
    b_cp0.wait()
    b_cp1.wait()

    coefs = [
        [plsc.load_gather(b_v, [jnp.full((L,), L + 4 * j + p, jnp.int32)])
         for p in range(Q + 1)]
        for j in range(Q + 1)
    ]

    def group_first_i(c, g):
        off = pl.multiple_of(c * CR + g * L, L)
        x = xs_v[pl.ds(off, L)]
        fi = (x / H).astype(jnp.int32)
        return x, fi

    def fill(buf, c):
        for g in range(GPC):
            x, fi = group_first_i(c, g)
            xm = x - fi.astype(jnp.float32) * H
            rows = g * L + lanes
            for j in range(Q + 1):
                xe = xm + _XE_OFF[j]
                cj = coefs[j]
                v = ((cj[3] * xe + cj[2]) * xe + cj[1]) * xe + cj[0]
                plsc.store_scatter(buf, [rows, fi + j], v)

    def unscatter(buf, c):
        for g in range(GPC):
            _, fi = group_first_i(c, g)
            rows = g * L + lanes
            for j in range(Q + 1):
                plsc.store_scatter(buf, [rows, fi + j], zero16)

    def start_dma(s, c):
        return pltpu.async_copy(
            bufs[s], out_hbm.at[pl.ds(row0 + c * CR, CR)], sems[s]
        )

    fill(buf0, 0)
    start_dma(0, 0)
    zero_buf(buf1)
    fill(buf1, 1)
    start_dma(1, 1)

    @pl.loop(1, NCHUNK // 2)
    def _(cc):
        for s in range(2):
            c = cc * 2 + s
            pltpu.make_async_copy(
                bufs[s], out_hbm.at[pl.ds(row0 + (c - 2) * CR, CR)], sems[s]
            ).wait()
            unscatter(bufs[s], c - 2)
            fill(bufs[s], c)
            start_dma(s, c)

    for s in range(2):
        pltpu.make_async_copy(
            bufs[s], out_hbm.at[pl.ds(row0 + (NCHUNK - 2 + s) * CR, CR)], sems[s]
        ).wait()


@jax.jit
def kernel(xs, B):
    return _sc_band(xs, B.reshape(-1))


# speedup vs baseline: 2.3340x; 1.0032x over previous
import functools

import numpy as np
import jax
import jax.numpy as jnp
from jax import lax
from jax.experimental import pallas as pl
from jax.experimental.pallas import tpu as pltpu
from jax.experimental.pallas import tpu_sc as plsc

H = 0.001
Q = 3
N_COLS = 1024
N_XS = 16384

NC, NS, L = 2, 16, 16
NW = NC * NS
RPW = N_XS // NW
CR = 16
NB = 4
NCHUNK = RPW // CR
GPC = CR // L

_XE_OFF = [float(np.float32(H) * np.float32(Q - j)) for j in range(Q + 1)]

_mesh = plsc.VectorSubcoreMesh(
    core_axis_name="c", subcore_axis_name="s", num_cores=NC, num_subcores=NS
)


@functools.partial(
    pl.kernel,
    out_type=jax.ShapeDtypeStruct((N_XS, N_COLS), jnp.float32),
    mesh=_mesh,
    scratch_types=[
        pltpu.VMEM((RPW,), jnp.float32),
        pltpu.VMEM((2 * L,), jnp.float32),
        pltpu.VMEM((CR, N_COLS), jnp.float32),
        pltpu.VMEM((CR, N_COLS), jnp.float32),
        pltpu.VMEM((CR, N_COLS), jnp.float32),
        pltpu.VMEM((CR, N_COLS), jnp.float32),
        pltpu.SemaphoreType.DMA,
        pltpu.SemaphoreType.DMA,
        pltpu.SemaphoreType.DMA,
        pltpu.SemaphoreType.DMA,
    ],
    compiler_params=pltpu.CompilerParams(needs_layout_passes=False),
)
def _sc_band(xs_hbm, b_hbm, out_hbm, xs_v, b_v,
             buf0, buf1, buf2, buf3, sem0, sem1, sem2, sem3):
    wid = lax.axis_index("s") * NC + lax.axis_index("c")
    row0 = wid * RPW
    xs_cp = pltpu.async_copy(xs_hbm.at[pl.ds(row0, RPW)], xs_v, sem0)
    b_cp0 = pltpu.async_copy(b_hbm, b_v.at[pl.ds(0, L)], sem1)
    b_cp1 = pltpu.async_copy(b_hbm, b_v.at[pl.ds(L, L)], sem1)

    lanes = lax.iota(jnp.int32, L)
    zero16 = jnp.zeros((L,), jnp.float32)
    bufs = (buf0, buf1, buf2, buf3)
    sems = (sem0, sem1, sem2, sem3)

    def zero_buf(buf):
        @pl.loop(0, CR)
        def _(r):
            for ci in range(N_COLS // L):
                buf[r, pl.ds(ci * L, L)] = zero16

    zero_buf(buf0)
    xs_cp.wait()
    b_cp0.wait()
    b_cp1.wait()

    coefs = [
        [plsc.load_gather(b_v, [jnp.full((L,), L + 4 * j + p, jnp.int32)])
         for p in range(Q + 1)]
        for j in range(Q + 1)
    ]

    def group_first_i(c, g):
        off = pl.multiple_of(c * CR + g * L, L)
        x = xs_v[pl.ds(off, L)]
        fi = (x / H).astype(jnp.int32)
        return x, fi

    def fill(buf, c):
        for g in range(GPC):
            x, fi = group_first_i(c, g)
            xm = x - fi.astype(jnp.float32) * H
            rows = g * L + lanes
            for j in range(Q + 1):
                xe = xm + _XE_OFF[j]
                cj = coefs[j]
                v = ((cj[3] * xe + cj[2]) * xe + cj[1]) * xe + cj[0]
                plsc.store_scatter(buf, [rows, fi + j], v)

    def unscatter(buf, c):
        for g in range(GPC):
            _, fi = group_first_i(c, g)
            rows = g * L + lanes
            for j in range(Q + 1):
                plsc.store_scatter(buf, [rows, fi + j], zero16)

    def start_dma(s, c):
        return pltpu.async_copy(
            bufs[s], out_hbm.at[pl.ds(row0 + c * CR, CR)], sems[s]
        )

    fill(buf0, 0)
    start_dma(0, 0)
    for s in range(1, NB):
        zero_buf(bufs[s])
        fill(bufs[s], s)
        start_dma(s, s)

    @pl.loop(1, NCHUNK // NB)
    def _(cc):
        for s in range(NB):
            c = cc * NB + s
            pltpu.make_async_copy(
                bufs[s], out_hbm.at[pl.ds(row0 + (c - NB) * CR, CR)], sems[s]
            ).wait()
            unscatter(bufs[s], c - NB)
            fill(bufs[s], c)
            start_dma(s, c)

    for s in range(NB):
        pltpu.make_async_copy(
            bufs[s], out_hbm.at[pl.ds(row0 + (NCHUNK - NB + s) * CR, CR)], sems[s]
        ).wait()


@jax.jit
def kernel(xs, B):
    return _sc_band(xs, B.reshape(-1))
